# baseline (speedup 1.0000x reference)
import functools

import jax
import jax.numpy as jnp
from jax import lax
from jax.experimental import pallas as pl
from jax.experimental.pallas import tpu as pltpu
from jax.experimental.pallas import tpu_sc as plsc

RADIUS = 0.15
NSAMPLE = 32
L = 16


def _f16(val):
    return jnp.full((L,), val, dtype=jnp.int32)


def _sc_grouper(new_xyz, xyz, cent, feat_rows, uxv, B, N, M, C, *,
                interpret=False):
    NW = 32
    QT = (B * M) // NW
    G = 4
    NG = QT // G
    NCHUNK = N // 128
    K = NSAMPLE
    CO = 3 + 2 * C
    r2 = jnp.float32(RADIUS * RADIUS)

    try:
        mesh = plsc.VectorSubcoreMesh(core_axis_name="c", subcore_axis_name="s")
    except ValueError:
        mesh = plsc.VectorSubcoreMesh(core_axis_name="c", subcore_axis_name="s",
                                      num_cores=2, num_subcores=16)

    def body(new_hbm, xyz_hbm, cent_hbm, rows_hbm, ux_hbm,
             of_hbm, ox_hbm,
             xyz_v, q_v, cent_v, ux_v, sel_v, idx_v, rows_v, obuf_v, xbuf_v,
             sem):
        cid = lax.axis_index("c")
        sid = lax.axis_index("s")
        wid = cid * 16 + sid
        wpb = NW // B
        b = wid // wpb
        m0 = (wid % wpb) * QT
        bN = b * N

        pltpu.sync_copy(xyz_hbm.at[b], xyz_v)
        pltpu.sync_copy(new_hbm.at[b, :, pl.ds(m0, QT)], q_v)
        pltpu.sync_copy(cent_hbm.at[b, :, pl.ds(m0, QT)], cent_v)
        pltpu.sync_copy(ux_hbm, ux_v)

        iota = lax.iota(jnp.int32, L)
        zeros16 = jnp.zeros((L,), jnp.int32)

        def per_query(q, carry):
            qx = plsc.load_gather(q_v, [zeros16, _f16(q)])
            qy = plsc.load_gather(q_v, [_f16(1), _f16(q)])
            qz = plsc.load_gather(q_v, [_f16(2), _f16(q)])

            def cond(c):
                c8, cntv = c
                return (c8 < NCHUNK) & (jnp.max(cntv) < K)

            def step(c):
                c8, cntv = c
                n0 = c8 * 128
                for s in range(8):
                    off = n0 + s * L
                    px = xyz_v[0, pl.ds(off, L)]
                    py = xyz_v[1, pl.ds(off, L)]
                    pz = xyz_v[2, pl.ds(off, L)]
                    dx = px - qx
                    dy = py - qy
                    dz = pz - qz
                    d2 = dx * dx + dy * dy + dz * dz
                    msk = d2 < r2
                    pc = plsc.all_reduce_population_count(msk)

                    @pl.when(jnp.max(pc) > 0)
                    def _():
                        pos = cntv + plsc.cumsum(msk.astype(jnp.int32)) - 1
                        pos = jnp.clip(pos, 0, 79)
                        plsc.store_scatter(sel_v, [pos], iota + off, mask=msk)

                    cntv = cntv + pc
                return c8 + 1, cntv

            _, cntv = lax.while_loop(cond, step, (0, zeros16))

            g = q // G
            col0 = (q % G) * K
            cur0 = sel_v[pl.ds(0, L)]
            first_s = jnp.min(jnp.where(iota == 0, cur0, jnp.int32(N)))
            firstv = jnp.full((L,), jnp.where(jnp.max(cntv) > 0, first_s, 0),
                              jnp.int32)
            selp0 = jnp.where(iota < cntv, cur0, firstv) + bN
            idx_v[g, pl.ds(col0, L)] = selp0
            kio = iota + L
            cur1 = sel_v[pl.ds(L, L)]
            selp1 = jnp.where(kio < cntv, cur1, firstv) + bN
            idx_v[g, pl.ds(col0 + L, L)] = selp1
            return carry

        lax.fori_loop(0, QT, per_query, 0)

        uxvec = ux_v[...]
        bNv = _f16(bN)

        def per_group(g, carry):
            pltpu.async_copy(rows_hbm.at[idx_v.at[g]], rows_v, sem).wait()

            def per_c(c, carry2):
                fc = _f16(c)
                for qin in range(G):
                    mq = g * G + qin
                    rb = qin * K
                    v0 = plsc.load_gather(rows_v, [_f16(rb) + iota, fc])
                    v1 = plsc.load_gather(rows_v, [_f16(rb + L) + iota, fc])
                    cf = plsc.load_gather(cent_v, [fc, _f16(mq)])
                    obuf_v[3 + c, qin, pl.ds(0, L)] = v0
                    obuf_v[3 + c, qin, pl.ds(L, L)] = v1
                    obuf_v[3 + C + c, qin, pl.ds(0, L)] = v0 - cf
                    obuf_v[3 + C + c, qin, pl.ds(L, L)] = v1 - cf
                return carry2

            lax.fori_loop(0, C, per_c, 0)

            for qin in range(G):
                mq = g * G + qin
                iv0 = idx_v[g, pl.ds(qin * K, L)] - bNv
                iv1 = idx_v[g, pl.ds(qin * K + L, L)] - bNv
                for d in range(3):
                    qd = plsc.load_gather(q_v, [_f16(d), _f16(mq)])
                    p0 = plsc.load_gather(xyz_v, [_f16(d), iv0]) - qd
                    p1 = plsc.load_gather(xyz_v, [_f16(d), iv1]) - qd
                    xbuf_v[d, qin, pl.ds(0, L)] = p0
                    xbuf_v[d, qin, pl.ds(L, L)] = p1
                    obuf_v[d, qin, pl.ds(0, L)] = p0 * uxvec
                    obuf_v[d, qin, pl.ds(L, L)] = p1 * uxvec

            mg = m0 + g * G
            pltpu.sync_copy(obuf_v, of_hbm.at[b, :, pl.ds(mg, G), :])
            pltpu.sync_copy(xbuf_v, ox_hbm.at[b, :, pl.ds(mg, G), :])
            return carry

        lax.fori_loop(0, NG, per_group, 0)

    grouper = pl.kernel(
        body,
        compiler_params=pltpu.CompilerParams(
            needs_layout_passes=False, use_tc_tiling_on_sc=False),
        out_type=(
            jax.ShapeDtypeStruct((B, CO, M, K), jnp.float32),
            jax.ShapeDtypeStruct((B, 3, M, K), jnp.float32),
        ),
        mesh=mesh,
        scratch_types=[
            pltpu.VMEM((3, N), jnp.float32),
            pltpu.VMEM((3, QT), jnp.float32),
            pltpu.VMEM((C, QT), jnp.float32),
            pltpu.VMEM((L,), jnp.float32),
            pltpu.VMEM((96,), jnp.int32),
            pltpu.VMEM((NG, G * K), jnp.int32),
            pltpu.VMEM((G * K, C), jnp.float32),
            pltpu.VMEM((CO, G, K), jnp.float32),
            pltpu.VMEM((3, G, K), jnp.float32),
            pltpu.SemaphoreType.DMA,
        ],
        interpret=interpret,
    )
    return grouper(new_xyz, xyz, cent, feat_rows, uxv)


def kernel(new_xyz, xyz, centroid_feature, feature, use_xyz):
    B, C, N = feature.shape
    M = new_xyz.shape[-1]
    feat_rows = jnp.transpose(feature, (0, 2, 1)).reshape(B * N, C)
    uxv = jnp.broadcast_to(
        jnp.where(jnp.asarray(use_xyz) != 0, jnp.float32(1), jnp.float32(0)),
        (L,))
    out_feat, out_xyz = _sc_grouper(new_xyz, xyz, centroid_feature,
                                    feat_rows, uxv, B, N, M, C)
    return (out_feat, out_xyz)


# speedup vs baseline: 1.3706x; 1.3706x over previous
import functools

import jax
import jax.numpy as jnp
from jax import lax
from jax.experimental import pallas as pl
from jax.experimental.pallas import tpu as pltpu
from jax.experimental.pallas import tpu_sc as plsc

RADIUS = 0.15
NSAMPLE = 32
L = 16


def _f16(val):
    return jnp.full((L,), val, dtype=jnp.int32)


def _sc_grouper(new_xyz, xyz, cent, feat_rows, uxv, B, N, M, C, *,
                interpret=False):
    NW = 32
    QT = (B * M) // NW
    G = 4
    NG = QT // G
    NCHUNK = N // 128
    K = NSAMPLE
    CO = 3 + 2 * C
    r2 = jnp.float32(RADIUS * RADIUS)

    try:
        mesh = plsc.VectorSubcoreMesh(core_axis_name="c", subcore_axis_name="s")
    except ValueError:
        mesh = plsc.VectorSubcoreMesh(core_axis_name="c", subcore_axis_name="s",
                                      num_cores=2, num_subcores=16)

    def body(new_hbm, xyz_hbm, cent_hbm, rows_hbm, ux_hbm,
             of_hbm, ox_hbm,
             xyz_v, q_v, cent_v, ux_v, sel_v, idx_v, rows_v, obuf_v, xbuf_v,
             sem):
        cid = lax.axis_index("c")
        sid = lax.axis_index("s")
        wid = cid * 16 + sid
        wpb = NW // B
        b = wid // wpb
        m0 = (wid % wpb) * QT
        bN = b * N

        pltpu.sync_copy(xyz_hbm.at[b], xyz_v)
        pltpu.sync_copy(new_hbm.at[b, :, pl.ds(m0, QT)], q_v)
        pltpu.sync_copy(cent_hbm.at[b, :, pl.ds(m0, QT)], cent_v)
        pltpu.sync_copy(ux_hbm, ux_v)

        iota = lax.iota(jnp.int32, L)
        zeros16 = jnp.zeros((L,), jnp.int32)

        def per_query(q, carry):
            qx = plsc.load_gather(q_v, [zeros16, _f16(q)])
            qy = plsc.load_gather(q_v, [_f16(1), _f16(q)])
            qz = plsc.load_gather(q_v, [_f16(2), _f16(q)])

            def cond(c):
                c8, cntv = c
                return (c8 < NCHUNK) & (jnp.max(cntv) < K)

            def step(c):
                c8, cntv = c
                n0 = c8 * 128
                for s in range(8):
                    off = n0 + s * L
                    px = xyz_v[0, pl.ds(off, L)]
                    py = xyz_v[1, pl.ds(off, L)]
                    pz = xyz_v[2, pl.ds(off, L)]
                    dx = px - qx
                    dy = py - qy
                    dz = pz - qz
                    d2 = dx * dx + dy * dy + dz * dz
                    msk = d2 < r2
                    pos = cntv + plsc.cumsum(msk.astype(jnp.int32)) - 1
                    pos = jnp.clip(pos, 0, 79)
                    plsc.store_scatter(sel_v, [pos], iota + off, mask=msk)
                    cntv = cntv + plsc.all_reduce_population_count(msk)
                return c8 + 1, cntv

            _, cntv = lax.while_loop(cond, step, (0, zeros16))

            g = q // G
            col0 = (q % G) * K
            cur0 = sel_v[pl.ds(0, L)]
            first_s = jnp.min(jnp.where(iota == 0, cur0, jnp.int32(N)))
            firstv = jnp.full((L,), jnp.where(jnp.max(cntv) > 0, first_s, 0),
                              jnp.int32)
            selp0 = jnp.where(iota < cntv, cur0, firstv) + bN
            idx_v[g, pl.ds(col0, L)] = selp0
            kio = iota + L
            cur1 = sel_v[pl.ds(L, L)]
            selp1 = jnp.where(kio < cntv, cur1, firstv) + bN
            idx_v[g, pl.ds(col0 + L, L)] = selp1
            return carry

        lax.fori_loop(0, QT, per_query, 0)

        uxvec = ux_v[...]
        bNv = _f16(bN)

        def per_group(g, carry):
            pltpu.async_copy(rows_hbm.at[idx_v.at[g]], rows_v, sem).wait()

            def per_c(c, carry2):
                fc = _f16(c)
                for qin in range(G):
                    mq = g * G + qin
                    rb = qin * K
                    v0 = plsc.load_gather(rows_v, [_f16(rb) + iota, fc])
                    v1 = plsc.load_gather(rows_v, [_f16(rb + L) + iota, fc])
                    cf = plsc.load_gather(cent_v, [fc, _f16(mq)])
                    obuf_v[3 + c, qin, pl.ds(0, L)] = v0
                    obuf_v[3 + c, qin, pl.ds(L, L)] = v1
                    obuf_v[3 + C + c, qin, pl.ds(0, L)] = v0 - cf
                    obuf_v[3 + C + c, qin, pl.ds(L, L)] = v1 - cf
                return carry2

            lax.fori_loop(0, C, per_c, 0)

            for qin in range(G):
                mq = g * G + qin
                iv0 = idx_v[g, pl.ds(qin * K, L)] - bNv
                iv1 = idx_v[g, pl.ds(qin * K + L, L)] - bNv
                for d in range(3):
                    qd = plsc.load_gather(q_v, [_f16(d), _f16(mq)])
                    p0 = plsc.load_gather(xyz_v, [_f16(d), iv0]) - qd
                    p1 = plsc.load_gather(xyz_v, [_f16(d), iv1]) - qd
                    xbuf_v[d, qin, pl.ds(0, L)] = p0
                    xbuf_v[d, qin, pl.ds(L, L)] = p1
                    obuf_v[d, qin, pl.ds(0, L)] = p0 * uxvec
                    obuf_v[d, qin, pl.ds(L, L)] = p1 * uxvec

            mg = m0 + g * G
            pltpu.sync_copy(obuf_v, of_hbm.at[b, :, pl.ds(mg, G), :])
            pltpu.sync_copy(xbuf_v, ox_hbm.at[b, :, pl.ds(mg, G), :])
            return carry

        lax.fori_loop(0, NG, per_group, 0)

    grouper = pl.kernel(
        body,
        compiler_params=pltpu.CompilerParams(
            needs_layout_passes=False, use_tc_tiling_on_sc=False),
        out_type=(
            jax.ShapeDtypeStruct((B, CO, M, K), jnp.float32),
            jax.ShapeDtypeStruct((B, 3, M, K), jnp.float32),
        ),
        mesh=mesh,
        scratch_types=[
            pltpu.VMEM((3, N), jnp.float32),
            pltpu.VMEM((3, QT), jnp.float32),
            pltpu.VMEM((C, QT), jnp.float32),
            pltpu.VMEM((L,), jnp.float32),
            pltpu.VMEM((96,), jnp.int32),
            pltpu.VMEM((NG, G * K), jnp.int32),
            pltpu.VMEM((G * K, C), jnp.float32),
            pltpu.VMEM((CO, G, K), jnp.float32),
            pltpu.VMEM((3, G, K), jnp.float32),
            pltpu.SemaphoreType.DMA,
        ],
        interpret=interpret,
    )
    return grouper(new_xyz, xyz, cent, feat_rows, uxv)


def kernel(new_xyz, xyz, centroid_feature, feature, use_xyz):
    B, C, N = feature.shape
    M = new_xyz.shape[-1]
    feat_rows = jnp.transpose(feature, (0, 2, 1)).reshape(B * N, C)
    uxv = jnp.broadcast_to(
        jnp.where(jnp.asarray(use_xyz) != 0, jnp.float32(1), jnp.float32(0)),
        (L,))
    out_feat, out_xyz = _sc_grouper(new_xyz, xyz, centroid_feature,
                                    feat_rows, uxv, B, N, M, C)
    return (out_feat, out_xyz)


# speedup vs baseline: 2.0724x; 1.5120x over previous
import functools

import jax
import jax.numpy as jnp
from jax import lax
from jax.experimental import pallas as pl
from jax.experimental.pallas import tpu as pltpu
from jax.experimental.pallas import tpu_sc as plsc

RADIUS = 0.15
NSAMPLE = 32
L = 16


def _f16(val):
    return jnp.full((L,), val, dtype=jnp.int32)


def _sc_grouper(new_xyz, xyz, cent, feat_rows, uxv, B, N, M, C, *,
                interpret=False):
    NW = 32
    QT = (B * M) // NW
    G = 4
    NG = QT // G
    NCHUNK = N // 128
    K = NSAMPLE
    CO = 3 + 2 * C
    r2 = jnp.float32(RADIUS * RADIUS)

    try:
        mesh = plsc.VectorSubcoreMesh(core_axis_name="c", subcore_axis_name="s")
    except ValueError:
        mesh = plsc.VectorSubcoreMesh(core_axis_name="c", subcore_axis_name="s",
                                      num_cores=2, num_subcores=16)

    def body(new_hbm, xyz_hbm, cent_hbm, rows_hbm, ux_hbm,
             of_hbm, ox_hbm,
             xyz_v, q_v, cent_v, ux_v, sel_v, idx_v, rows_v, obuf_v, xbuf_v,
             sem):
        cid = lax.axis_index("c")
        sid = lax.axis_index("s")
        wid = cid * 16 + sid
        wpb = NW // B
        b = wid // wpb
        m0 = (wid % wpb) * QT
        bN = b * N

        pltpu.sync_copy(xyz_hbm.at[b], xyz_v)
        pltpu.sync_copy(new_hbm.at[b, :, pl.ds(m0, QT)], q_v)
        pltpu.sync_copy(cent_hbm.at[b, :, pl.ds(m0, QT)], cent_v)
        pltpu.sync_copy(ux_hbm, ux_v)

        iota = lax.iota(jnp.int32, L)
        zeros16 = jnp.zeros((L,), jnp.int32)

        def per_query(q, carry):
            qx = plsc.load_gather(q_v, [zeros16, _f16(q)])
            qy = plsc.load_gather(q_v, [_f16(1), _f16(q)])
            qz = plsc.load_gather(q_v, [_f16(2), _f16(q)])

            def cond(c):
                c8, cntv = c
                return (c8 < NCHUNK) & (jnp.max(cntv) < K)

            def step(c):
                c8, cntv = c
                n0 = c8 * 128
                msks = []
                for s in range(8):
                    off = n0 + s * L
                    px = xyz_v[0, pl.ds(off, L)]
                    py = xyz_v[1, pl.ds(off, L)]
                    pz = xyz_v[2, pl.ds(off, L)]
                    dx = px - qx
                    dy = py - qy
                    dz = pz - qz
                    d2 = dx * dx + dy * dy + dz * dz
                    msks.append(d2 < r2)
                pcs = [plsc.all_reduce_population_count(m) for m in msks]
                bases = []
                for s in range(8):
                    bases.append(cntv)
                    cntv = cntv + pcs[s]
                for s in range(8):
                    pos = bases[s] + plsc.cumsum(msks[s].astype(jnp.int32)) - 1
                    pos = jnp.clip(pos, 0, 79)
                    plsc.store_scatter(sel_v, [pos], iota + (n0 + s * L),
                                       mask=msks[s])
                return c8 + 1, cntv

            _, cntv = lax.while_loop(cond, step, (0, zeros16))

            g = q // G
            col0 = (q % G) * K
            cur0 = sel_v[pl.ds(0, L)]
            first_s = jnp.min(jnp.where(iota == 0, cur0, jnp.int32(N)))
            firstv = jnp.full((L,), jnp.where(jnp.max(cntv) > 0, first_s, 0),
                              jnp.int32)
            selp0 = jnp.where(iota < cntv, cur0, firstv) + bN
            idx_v[g, pl.ds(col0, L)] = selp0
            kio = iota + L
            cur1 = sel_v[pl.ds(L, L)]
            selp1 = jnp.where(kio < cntv, cur1, firstv) + bN
            idx_v[g, pl.ds(col0 + L, L)] = selp1
            return carry

        lax.fori_loop(0, QT, per_query, 0)

        uxvec = ux_v[...]
        bNv = _f16(bN)

        def per_group(g, carry):
            pltpu.async_copy(rows_hbm.at[idx_v.at[g]], rows_v, sem).wait()

            def per_c(c, carry2):
                fc = _f16(c)
                for qin in range(G):
                    mq = g * G + qin
                    rb = qin * K
                    v0 = plsc.load_gather(rows_v, [_f16(rb) + iota, fc])
                    v1 = plsc.load_gather(rows_v, [_f16(rb + L) + iota, fc])
                    cf = plsc.load_gather(cent_v, [fc, _f16(mq)])
                    obuf_v[3 + c, qin, pl.ds(0, L)] = v0
                    obuf_v[3 + c, qin, pl.ds(L, L)] = v1
                    obuf_v[3 + C + c, qin, pl.ds(0, L)] = v0 - cf
                    obuf_v[3 + C + c, qin, pl.ds(L, L)] = v1 - cf
                return carry2

            lax.fori_loop(0, C, per_c, 0)

            for qin in range(G):
                mq = g * G + qin
                iv0 = idx_v[g, pl.ds(qin * K, L)] - bNv
                iv1 = idx_v[g, pl.ds(qin * K + L, L)] - bNv
                for d in range(3):
                    qd = plsc.load_gather(q_v, [_f16(d), _f16(mq)])
                    p0 = plsc.load_gather(xyz_v, [_f16(d), iv0]) - qd
                    p1 = plsc.load_gather(xyz_v, [_f16(d), iv1]) - qd
                    xbuf_v[d, qin, pl.ds(0, L)] = p0
                    xbuf_v[d, qin, pl.ds(L, L)] = p1
                    obuf_v[d, qin, pl.ds(0, L)] = p0 * uxvec
                    obuf_v[d, qin, pl.ds(L, L)] = p1 * uxvec

            mg = m0 + g * G
            pltpu.sync_copy(obuf_v, of_hbm.at[b, :, pl.ds(mg, G), :])
            pltpu.sync_copy(xbuf_v, ox_hbm.at[b, :, pl.ds(mg, G), :])
            return carry

        lax.fori_loop(0, NG, per_group, 0)

    grouper = pl.kernel(
        body,
        compiler_params=pltpu.CompilerParams(
            needs_layout_passes=False, use_tc_tiling_on_sc=False),
        out_type=(
            jax.ShapeDtypeStruct((B, CO, M, K), jnp.float32),
            jax.ShapeDtypeStruct((B, 3, M, K), jnp.float32),
        ),
        mesh=mesh,
        scratch_types=[
            pltpu.VMEM((3, N), jnp.float32),
            pltpu.VMEM((3, QT), jnp.float32),
            pltpu.VMEM((C, QT), jnp.float32),
            pltpu.VMEM((L,), jnp.float32),
            pltpu.VMEM((96,), jnp.int32),
            pltpu.VMEM((NG, G * K), jnp.int32),
            pltpu.VMEM((G * K, C), jnp.float32),
            pltpu.VMEM((CO, G, K), jnp.float32),
            pltpu.VMEM((3, G, K), jnp.float32),
            pltpu.SemaphoreType.DMA,
        ],
        interpret=interpret,
    )
    return grouper(new_xyz, xyz, cent, feat_rows, uxv)


def kernel(new_xyz, xyz, centroid_feature, feature, use_xyz):
    B, C, N = feature.shape
    M = new_xyz.shape[-1]
    feat_rows = jnp.transpose(feature, (0, 2, 1)).reshape(B * N, C)
    uxv = jnp.broadcast_to(
        jnp.where(jnp.asarray(use_xyz) != 0, jnp.float32(1), jnp.float32(0)),
        (L,))
    out_feat, out_xyz = _sc_grouper(new_xyz, xyz, centroid_feature,
                                    feat_rows, uxv, B, N, M, C)
    return (out_feat, out_xyz)


# speedup vs baseline: 2.2274x; 1.0748x over previous
import functools

import jax
import jax.numpy as jnp
from jax import lax
from jax.experimental import pallas as pl
from jax.experimental.pallas import tpu as pltpu
from jax.experimental.pallas import tpu_sc as plsc

RADIUS = 0.15
NSAMPLE = 32
L = 16


def _f16(val):
    return jnp.full((L,), val, dtype=jnp.int32)


def _sc_grouper(new_xyz, xyz, cent, feat_rows, uxv, B, N, M, C, *,
                interpret=False):
    NW = 32
    QT = (B * M) // NW
    G = 4
    NG = QT // G
    NCHUNK = N // 128
    K = NSAMPLE
    CO = 3 + 2 * C
    r2 = jnp.float32(RADIUS * RADIUS)

    try:
        mesh = plsc.VectorSubcoreMesh(core_axis_name="c", subcore_axis_name="s")
    except ValueError:
        mesh = plsc.VectorSubcoreMesh(core_axis_name="c", subcore_axis_name="s",
                                      num_cores=2, num_subcores=16)

    def body(new_hbm, xyz_hbm, cent_hbm, rows_hbm, ux_hbm,
             of_hbm, ox_hbm,
             xyz_v, q_v, cent_v, ux_v, sel_v, idx_v, rows_v, obuf_v, xbuf_v,
             gsem0, gsem1, osem0, osem1):
        gsem = [gsem0, gsem1]
        osem = [osem0, osem1]
        cid = lax.axis_index("c")
        sid = lax.axis_index("s")
        wid = cid * 16 + sid
        wpb = NW // B
        b = wid // wpb
        m0 = (wid % wpb) * QT
        bN = b * N

        pltpu.sync_copy(xyz_hbm.at[b], xyz_v)
        pltpu.sync_copy(new_hbm.at[b, :, pl.ds(m0, QT)], q_v)
        pltpu.sync_copy(cent_hbm.at[b, :, pl.ds(m0, QT)], cent_v)
        pltpu.sync_copy(ux_hbm, ux_v)

        iota = lax.iota(jnp.int32, L)
        zeros16 = jnp.zeros((L,), jnp.int32)

        def per_query(q, carry):
            qx = plsc.load_gather(q_v, [zeros16, _f16(q)])
            qy = plsc.load_gather(q_v, [_f16(1), _f16(q)])
            qz = plsc.load_gather(q_v, [_f16(2), _f16(q)])

            def cond(c):
                c8, cntv = c
                return (c8 < NCHUNK) & (jnp.max(cntv) < K)

            def step(c):
                c8, cntv = c
                n0 = c8 * 128
                msks = []
                for s in range(8):
                    off = n0 + s * L
                    px = xyz_v[0, pl.ds(off, L)]
                    py = xyz_v[1, pl.ds(off, L)]
                    pz = xyz_v[2, pl.ds(off, L)]
                    dx = px - qx
                    dy = py - qy
                    dz = pz - qz
                    d2 = dx * dx + dy * dy + dz * dz
                    msks.append(d2 < r2)
                pcs = [plsc.all_reduce_population_count(m) for m in msks]
                bases = []
                for s in range(8):
                    bases.append(cntv)
                    cntv = cntv + pcs[s]
                for s in range(8):
                    pos = bases[s] + plsc.cumsum(msks[s].astype(jnp.int32)) - 1
                    pos = jnp.clip(pos, 0, 79)
                    plsc.store_scatter(sel_v, [pos], iota + (n0 + s * L),
                                       mask=msks[s])
                return c8 + 1, cntv

            _, cntv = lax.while_loop(cond, step, (0, zeros16))

            g = q // G
            col0 = (q % G) * K
            cur0 = sel_v[pl.ds(0, L)]
            first_s = jnp.min(jnp.where(iota == 0, cur0, jnp.int32(N)))
            firstv = jnp.full((L,), jnp.where(jnp.max(cntv) > 0, first_s, 0),
                              jnp.int32)
            selp0 = jnp.where(iota < cntv, cur0, firstv) + bN
            idx_v[g, pl.ds(col0, L)] = selp0
            kio = iota + L
            cur1 = sel_v[pl.ds(L, L)]
            selp1 = jnp.where(kio < cntv, cur1, firstv) + bN
            idx_v[g, pl.ds(col0 + L, L)] = selp1
            return carry

        lax.fori_loop(0, QT, per_query, 0)

        uxvec = ux_v[...]
        bNv = _f16(bN)

        def compute_group(g, rbuf, ob, xb):
            def per_c(c, carry2):
                fc = _f16(c)
                for qin in range(G):
                    mq = g * G + qin
                    rb = qin * K
                    v0 = plsc.load_gather(rbuf, [_f16(rb) + iota, fc])
                    v1 = plsc.load_gather(rbuf, [_f16(rb + L) + iota, fc])
                    cf = plsc.load_gather(cent_v, [fc, _f16(mq)])
                    ob[3 + c, qin, pl.ds(0, L)] = v0
                    ob[3 + c, qin, pl.ds(L, L)] = v1
                    ob[3 + C + c, qin, pl.ds(0, L)] = v0 - cf
                    ob[3 + C + c, qin, pl.ds(L, L)] = v1 - cf
                return carry2

            lax.fori_loop(0, C, per_c, 0)

            for qin in range(G):
                mq = g * G + qin
                iv0 = idx_v[g, pl.ds(qin * K, L)] - bNv
                iv1 = idx_v[g, pl.ds(qin * K + L, L)] - bNv
                for d in range(3):
                    qd = plsc.load_gather(q_v, [_f16(d), _f16(mq)])
                    p0 = plsc.load_gather(xyz_v, [_f16(d), iv0]) - qd
                    p1 = plsc.load_gather(xyz_v, [_f16(d), iv1]) - qd
                    xb[d, qin, pl.ds(0, L)] = p0
                    xb[d, qin, pl.ds(L, L)] = p1
                    ob[d, qin, pl.ds(0, L)] = p0 * uxvec
                    ob[d, qin, pl.ds(L, L)] = p1 * uxvec

        for bslot in range(2):
            pltpu.async_copy(rows_hbm.at[idx_v.at[bslot]],
                             rows_v.at[bslot], gsem[bslot])

        def per_pair(i, carry):
            for bslot in range(2):
                g = i * 2 + bslot
                rbuf = rows_v.at[bslot]
                ob = obuf_v.at[bslot]
                xb = xbuf_v.at[bslot]
                mg = m0 + g * G
                of_slab = of_hbm.at[b, :, pl.ds(mg, G), :]
                ox_slab = ox_hbm.at[b, :, pl.ds(mg, G), :]
                pltpu.make_async_copy(rows_hbm.at[idx_v.at[g]], rbuf,
                                      gsem[bslot]).wait()

                @pl.when(i > 0)
                def _():
                    pltpu.make_async_copy(ob, of_slab, osem[bslot]).wait()
                    pltpu.make_async_copy(xb, ox_slab, osem[bslot]).wait()

                compute_group(g, rbuf, ob, xb)

                pltpu.async_copy(ob, of_slab, osem[bslot])
                pltpu.async_copy(xb, ox_slab, osem[bslot])

                @pl.when(i < NG // 2 - 1)
                def _():
                    pltpu.async_copy(rows_hbm.at[idx_v.at[g + 2]], rbuf,
                                     gsem[bslot])
            return carry

        lax.fori_loop(0, NG // 2, per_pair, 0)

        for bslot in range(2):
            g = NG - 2 + bslot
            mg = m0 + g * G
            pltpu.make_async_copy(obuf_v.at[bslot],
                                  of_hbm.at[b, :, pl.ds(mg, G), :],
                                  osem[bslot]).wait()
            pltpu.make_async_copy(xbuf_v.at[bslot],
                                  ox_hbm.at[b, :, pl.ds(mg, G), :],
                                  osem[bslot]).wait()

    grouper = pl.kernel(
        body,
        compiler_params=pltpu.CompilerParams(
            needs_layout_passes=False, use_tc_tiling_on_sc=False),
        out_type=(
            jax.ShapeDtypeStruct((B, CO, M, K), jnp.float32),
            jax.ShapeDtypeStruct((B, 3, M, K), jnp.float32),
        ),
        mesh=mesh,
        scratch_types=[
            pltpu.VMEM((3, N), jnp.float32),
            pltpu.VMEM((3, QT), jnp.float32),
            pltpu.VMEM((C, QT), jnp.float32),
            pltpu.VMEM((L,), jnp.float32),
            pltpu.VMEM((96,), jnp.int32),
            pltpu.VMEM((NG, G * K), jnp.int32),
            pltpu.VMEM((2, G * K, C), jnp.float32),
            pltpu.VMEM((2, CO, G, K), jnp.float32),
            pltpu.VMEM((2, 3, G, K), jnp.float32),
            pltpu.SemaphoreType.DMA,
            pltpu.SemaphoreType.DMA,
            pltpu.SemaphoreType.DMA,
            pltpu.SemaphoreType.DMA,
        ],
        interpret=interpret,
    )
    return grouper(new_xyz, xyz, cent, feat_rows, uxv)


def kernel(new_xyz, xyz, centroid_feature, feature, use_xyz):
    B, C, N = feature.shape
    M = new_xyz.shape[-1]
    feat_rows = jnp.transpose(feature, (0, 2, 1)).reshape(B * N, C)
    uxv = jnp.broadcast_to(
        jnp.where(jnp.asarray(use_xyz) != 0, jnp.float32(1), jnp.float32(0)),
        (L,))
    out_feat, out_xyz = _sc_grouper(new_xyz, xyz, centroid_feature,
                                    feat_rows, uxv, B, N, M, C)
    return (out_feat, out_xyz)
